# baseline (speedup 1.0000x reference)
import functools

import jax
import jax.numpy as jnp
from jax import lax
from jax.experimental import pallas as pl
from jax.experimental.pallas import tpu as pltpu
from jax.experimental.pallas import tpu_sc as plsc

B = 16384
D = 64
NC = 2
NS = 16
NW = NC * NS
BW = B // NW
L = 16


def _sc_gather_partials(uidx, fidx, uembT, ubiasT, fembT, fbiasT):
    mesh = plsc.VectorSubcoreMesh(core_axis_name="c", subcore_axis_name="s")

    @functools.partial(
        pl.kernel,
        mesh=mesh,
        compiler_params=pltpu.CompilerParams(use_tc_tiling_on_sc=False),
        out_type=(
            jax.ShapeDtypeStruct((NW * L,), jnp.float32),
            jax.ShapeDtypeStruct((B,), jnp.float32),
            jax.ShapeDtypeStruct((B,), jnp.float32),
        ),
        scratch_types=[
            pltpu.VMEM((BW,), jnp.int32),
            pltpu.VMEM((BW,), jnp.int32),
            pltpu.VMEM((D, BW), jnp.float32),
            pltpu.VMEM((D, BW), jnp.float32),
            pltpu.VMEM((BW,), jnp.float32),
            pltpu.VMEM((BW,), jnp.float32),
            pltpu.VMEM((L,), jnp.float32),
            pltpu.SemaphoreType.DMA,
            pltpu.SemaphoreType.DMA,
            pltpu.SemaphoreType.DMA,
            pltpu.SemaphoreType.DMA,
        ],
    )
    def k(uidx_hbm, fidx_hbm, uembT_hbm, ubiasT_hbm, fembT_hbm, fbiasT_hbm,
          part_hbm, ub_hbm, fb_hbm,
          uidx_v, fidx_v, urows_v, frows_v, ub_v, fb_v, part_v,
          sem_u, sem_f, sem_ub, sem_fb):
        wid = lax.axis_index("s") * NC + lax.axis_index("c")
        base = wid * BW
        pltpu.sync_copy(uidx_hbm.at[pl.ds(base, BW)], uidx_v)
        pltpu.sync_copy(fidx_hbm.at[pl.ds(base, BW)], fidx_v)
        cub = pltpu.async_copy(ubiasT_hbm.at[0].at[uidx_v], ub_v, sem_ub)
        cfb = pltpu.async_copy(fbiasT_hbm.at[0].at[fidx_v], fb_v, sem_fb)

        pend = []
        for kk in range(D):
            cu = pltpu.async_copy(
                uembT_hbm.at[kk].at[uidx_v], urows_v.at[kk], sem_u)
            cf = pltpu.async_copy(
                fembT_hbm.at[kk].at[fidx_v], frows_v.at[kk], sem_f)
            pend.append(cu)
            pend.append(cf)
        for c in pend:
            c.wait()

        zero = jnp.zeros((L,), jnp.float32)

        def dim_body(kk, accs):
            a0, a1, a2, a3 = accs
            for j in range(0, BW // L, 4):
                a0 = a0 + (urows_v[kk, pl.ds((j + 0) * L, L)]
                           * frows_v[kk, pl.ds((j + 0) * L, L)])
                a1 = a1 + (urows_v[kk, pl.ds((j + 1) * L, L)]
                           * frows_v[kk, pl.ds((j + 1) * L, L)])
                a2 = a2 + (urows_v[kk, pl.ds((j + 2) * L, L)]
                           * frows_v[kk, pl.ds((j + 2) * L, L)])
                a3 = a3 + (urows_v[kk, pl.ds((j + 3) * L, L)]
                           * frows_v[kk, pl.ds((j + 3) * L, L)])
            return (a0, a1, a2, a3)

        a0, a1, a2, a3 = lax.fori_loop(0, D, dim_body, (zero, zero, zero, zero))
        part_v[...] = (a0 + a1) + (a2 + a3)
        pltpu.sync_copy(part_v, part_hbm.at[pl.ds(wid * L, L)])
        cub.wait()
        cfb.wait()
        pltpu.sync_copy(ub_v, ub_hbm.at[pl.ds(base, BW)])
        pltpu.sync_copy(fb_v, fb_hbm.at[pl.ds(base, BW)])

    return k(uidx, fidx, uembT, ubiasT, fembT, fbiasT)


def _tc_finish(part, ub, fb):
    def body(p_ref, u_ref, f_ref, o_ref):
        s = jnp.sum(p_ref[...])
        o_ref[...] = jax.nn.sigmoid(u_ref[...] + f_ref[...] + s)

    return pl.pallas_call(
        body,
        out_shape=jax.ShapeDtypeStruct((128, 128), jnp.float32),
    )(part.reshape(4, 128), ub.reshape(128, 128), fb.reshape(128, 128))


def kernel(inputs, user_emb, user_bias, food_emb, food_bias):
    uidx = inputs[:, 0]
    fidx = inputs[:, 1]
    part, ub, fb = _sc_gather_partials(
        uidx, fidx, user_emb.T, user_bias.T, food_emb.T, food_bias.T)
    return _tc_finish(part, ub, fb).reshape(B, 1)


# speedup vs baseline: 22.8823x; 22.8823x over previous
import functools

import jax
import jax.numpy as jnp
from jax import lax
from jax.experimental import pallas as pl
from jax.experimental.pallas import tpu as pltpu
from jax.experimental.pallas import tpu_sc as plsc

B = 16384
D = 64
NC = 2
NS = 16
NW = NC * NS
BW = B // NW
L = 16


def _sc_detile(uembT, fembT):
    mesh = plsc.VectorSubcoreMesh(core_axis_name="c", subcore_axis_name="s")

    NP = 16
    PLEN = 62464
    PLAST = 1000000 - 15 * PLEN
    pieces = [(p * PLEN, PLEN if p < NP - 1 else PLAST) for p in range(NP)]

    @functools.partial(
        pl.kernel,
        mesh=mesh,
        out_type=(
            jax.ShapeDtypeStruct((D * 1000000,), jnp.float32),
            jax.ShapeDtypeStruct((D * 1000000,), jnp.float32),
        ),
        scratch_types=[
            pltpu.VMEM((PLAST,), jnp.float32),
            pltpu.VMEM((PLAST,), jnp.float32),
            pltpu.SemaphoreType.DMA,
            pltpu.SemaphoreType.DMA,
            pltpu.SemaphoreType.DMA,
            pltpu.SemaphoreType.DMA,
        ],
    )
    def k(uembT_hbm, fembT_hbm, ulin_hbm, flin_hbm,
          buf0, buf1, rs0, rs1, ws0, ws1):
        wid = lax.axis_index("s") * NC + lax.axis_index("c")
        bufs = (buf0, buf1)
        rsems = (rs0, rs1)
        wsems = (ws0, ws1)
        jobs = []
        for r in range(2):
            jobs += [("u", r, p) for p in range(NP)]
            jobs += [("f", r, p) for p in range(NP)]

        def fire_read(j):
            tbl, r, p = jobs[j]
            src = uembT_hbm if tbl == "u" else fembT_hbm
            off, ln = pieces[p]
            b = j % 2
            return pltpu.async_copy(
                src.at[wid * 2 + r].at[pl.ds(off, ln)],
                bufs[b].at[pl.ds(0, ln)], rsems[b])

        def fire_write(j):
            tbl, r, p = jobs[j]
            dst = ulin_hbm if tbl == "u" else flin_hbm
            off, ln = pieces[p]
            b = j % 2
            return pltpu.async_copy(
                bufs[b].at[pl.ds(0, ln)],
                dst.at[pl.ds((wid * 2 + r) * 1000000 + off, ln)], wsems[b])

        nj = len(jobs)
        rd = [None] * nj
        wr = [None] * nj
        rd[0] = fire_read(0)
        rd[1] = fire_read(1)
        for j in range(nj):
            rd[j].wait()
            wr[j] = fire_write(j)
            if j + 2 < nj:
                wr[j].wait()
                rd[j + 2] = fire_read(j + 2)
        wr[nj - 2].wait()
        wr[nj - 1].wait()

    return k(uembT, fembT)


def _sc_gather_partials(uidx, fidx, uembT, ubiasT, fembT, fbiasT):
    mesh = plsc.VectorSubcoreMesh(core_axis_name="c", subcore_axis_name="s")

    @functools.partial(
        pl.kernel,
        mesh=mesh,
        compiler_params=pltpu.CompilerParams(use_tc_tiling_on_sc=False),
        out_type=(
            jax.ShapeDtypeStruct((NW * L,), jnp.float32),
            jax.ShapeDtypeStruct((B,), jnp.float32),
            jax.ShapeDtypeStruct((B,), jnp.float32),
        ),
        scratch_types=[
            pltpu.VMEM((BW,), jnp.int32),
            pltpu.VMEM((BW,), jnp.int32),
            pltpu.VMEM((D, BW), jnp.float32),
            pltpu.VMEM((D, BW), jnp.float32),
            pltpu.VMEM((BW,), jnp.float32),
            pltpu.VMEM((BW,), jnp.float32),
            pltpu.VMEM((L,), jnp.float32),
            pltpu.SemaphoreType.DMA,
            pltpu.SemaphoreType.DMA,
            pltpu.SemaphoreType.DMA,
            pltpu.SemaphoreType.DMA,
        ],
    )
    def k(uidx_hbm, fidx_hbm, uembT_hbm, ubiasT_hbm, fembT_hbm, fbiasT_hbm,
          part_hbm, ub_hbm, fb_hbm,
          uidx_v, fidx_v, urows_v, frows_v, ub_v, fb_v, part_v,
          sem_u, sem_f, sem_ub, sem_fb):
        wid = lax.axis_index("s") * NC + lax.axis_index("c")
        base = wid * BW
        pltpu.sync_copy(uidx_hbm.at[pl.ds(base, BW)], uidx_v)
        pltpu.sync_copy(fidx_hbm.at[pl.ds(base, BW)], fidx_v)
        cub = pltpu.async_copy(ubiasT_hbm.at[0].at[uidx_v], ub_v, sem_ub)
        cfb = pltpu.async_copy(fbiasT_hbm.at[0].at[fidx_v], fb_v, sem_fb)

        pend = []
        for kk in range(D):
            cu = pltpu.async_copy(
                uembT_hbm.at[kk].at[uidx_v], urows_v.at[kk], sem_u)
            cf = pltpu.async_copy(
                fembT_hbm.at[kk].at[fidx_v], frows_v.at[kk], sem_f)
            pend.append(cu)
            pend.append(cf)
        for c in pend:
            c.wait()

        zero = jnp.zeros((L,), jnp.float32)

        def dim_body(kk, accs):
            a0, a1, a2, a3 = accs
            for j in range(0, BW // L, 4):
                a0 = a0 + (urows_v[kk, pl.ds((j + 0) * L, L)]
                           * frows_v[kk, pl.ds((j + 0) * L, L)])
                a1 = a1 + (urows_v[kk, pl.ds((j + 1) * L, L)]
                           * frows_v[kk, pl.ds((j + 1) * L, L)])
                a2 = a2 + (urows_v[kk, pl.ds((j + 2) * L, L)]
                           * frows_v[kk, pl.ds((j + 2) * L, L)])
                a3 = a3 + (urows_v[kk, pl.ds((j + 3) * L, L)]
                           * frows_v[kk, pl.ds((j + 3) * L, L)])
            return (a0, a1, a2, a3)

        a0, a1, a2, a3 = lax.fori_loop(0, D, dim_body, (zero, zero, zero, zero))
        part_v[...] = (a0 + a1) + (a2 + a3)
        pltpu.sync_copy(part_v, part_hbm.at[pl.ds(wid * L, L)])
        cub.wait()
        cfb.wait()
        pltpu.sync_copy(ub_v, ub_hbm.at[pl.ds(base, BW)])
        pltpu.sync_copy(fb_v, fb_hbm.at[pl.ds(base, BW)])

    return k(uidx, fidx, uembT, ubiasT, fembT, fbiasT)


def _tc_finish(part, ub, fb):
    def body(p_ref, u_ref, f_ref, o_ref):
        s = jnp.sum(p_ref[...])
        o_ref[...] = jax.nn.sigmoid(u_ref[...] + f_ref[...] + s)

    return pl.pallas_call(
        body,
        out_shape=jax.ShapeDtypeStruct((128, 128), jnp.float32),
    )(part.reshape(4, 128), ub.reshape(128, 128), fb.reshape(128, 128))


def kernel(inputs, user_emb, user_bias, food_emb, food_bias):
    uidx = inputs[:, 0]
    fidx = inputs[:, 1]
    ulin, flin = _sc_detile(user_emb.T, food_emb.T)
    part, ub, fb = _sc_gather_partials(
        uidx, fidx, ulin.reshape(D, 1000000), user_bias.T,
        flin.reshape(D, 1000000), food_bias.T)
    return _tc_finish(part, ub, fb).reshape(B, 1)


# speedup vs baseline: 22.9085x; 1.0011x over previous
import functools

import jax
import jax.numpy as jnp
from jax import lax
from jax.experimental import pallas as pl
from jax.experimental.pallas import tpu as pltpu
from jax.experimental.pallas import tpu_sc as plsc

B = 16384
D = 64
NC = 2
NS = 16
NW = NC * NS
BW = B // NW
L = 16


def _sc_detile(uembT, fembT):
    mesh = plsc.VectorSubcoreMesh(core_axis_name="c", subcore_axis_name="s")

    NB = 4
    NP = 32
    PLEN = 31232
    PLAST = 1000000 - 31 * PLEN
    pieces = [(p * PLEN, PLEN if p < NP - 1 else PLAST) for p in range(NP)]

    @functools.partial(
        pl.kernel,
        mesh=mesh,
        out_type=(
            jax.ShapeDtypeStruct((D * 1000000,), jnp.float32),
            jax.ShapeDtypeStruct((D * 1000000,), jnp.float32),
        ),
        scratch_types=(
            [pltpu.VMEM((PLAST,), jnp.float32) for _ in range(NB)]
            + [pltpu.SemaphoreType.DMA for _ in range(2 * NB)]
        ),
    )
    def k(uembT_hbm, fembT_hbm, ulin_hbm, flin_hbm, *scr):
        bufs = scr[:NB]
        rsems = scr[NB:2 * NB]
        wsems = scr[2 * NB:]
        wid = lax.axis_index("s") * NC + lax.axis_index("c")
        jobs = []
        for r in range(2):
            jobs += [("u", r, p) for p in range(NP)]
            jobs += [("f", r, p) for p in range(NP)]

        def fire_read(j):
            tbl, r, p = jobs[j]
            src = uembT_hbm if tbl == "u" else fembT_hbm
            off, ln = pieces[p]
            b = j % NB
            return pltpu.async_copy(
                src.at[wid * 2 + r].at[pl.ds(off, ln)],
                bufs[b].at[pl.ds(0, ln)], rsems[b])

        def fire_write(j):
            tbl, r, p = jobs[j]
            dst = ulin_hbm if tbl == "u" else flin_hbm
            off, ln = pieces[p]
            b = j % NB
            return pltpu.async_copy(
                bufs[b].at[pl.ds(0, ln)],
                dst.at[pl.ds((wid * 2 + r) * 1000000 + off, ln)], wsems[b])

        nj = len(jobs)
        rd = [None] * nj
        wr = [None] * nj
        for j in range(NB):
            rd[j] = fire_read(j)
        for j in range(nj):
            rd[j].wait()
            wr[j] = fire_write(j)
            if j + NB < nj:
                wr[j].wait()
                rd[j + NB] = fire_read(j + NB)
        for j in range(max(0, nj - NB), nj):
            if wr[j] is not None and j + NB >= nj:
                wr[j].wait()

    return k(uembT, fembT)


def _sc_gather_partials(uidx, fidx, uembT, ubiasT, fembT, fbiasT):
    mesh = plsc.VectorSubcoreMesh(core_axis_name="c", subcore_axis_name="s")

    @functools.partial(
        pl.kernel,
        mesh=mesh,
        compiler_params=pltpu.CompilerParams(use_tc_tiling_on_sc=False),
        out_type=(
            jax.ShapeDtypeStruct((NW * L,), jnp.float32),
            jax.ShapeDtypeStruct((B,), jnp.float32),
            jax.ShapeDtypeStruct((B,), jnp.float32),
        ),
        scratch_types=[
            pltpu.VMEM((BW,), jnp.int32),
            pltpu.VMEM((BW,), jnp.int32),
            pltpu.VMEM((D, BW), jnp.float32),
            pltpu.VMEM((D, BW), jnp.float32),
            pltpu.VMEM((BW,), jnp.float32),
            pltpu.VMEM((BW,), jnp.float32),
            pltpu.VMEM((L,), jnp.float32),
            pltpu.SemaphoreType.DMA,
            pltpu.SemaphoreType.DMA,
            pltpu.SemaphoreType.DMA,
            pltpu.SemaphoreType.DMA,
        ],
    )
    def k(uidx_hbm, fidx_hbm, uembT_hbm, ubiasT_hbm, fembT_hbm, fbiasT_hbm,
          part_hbm, ub_hbm, fb_hbm,
          uidx_v, fidx_v, urows_v, frows_v, ub_v, fb_v, part_v,
          sem_u, sem_f, sem_ub, sem_fb):
        wid = lax.axis_index("s") * NC + lax.axis_index("c")
        base = wid * BW
        pltpu.sync_copy(uidx_hbm.at[pl.ds(base, BW)], uidx_v)
        pltpu.sync_copy(fidx_hbm.at[pl.ds(base, BW)], fidx_v)
        cub = pltpu.async_copy(ubiasT_hbm.at[0].at[uidx_v], ub_v, sem_ub)
        cfb = pltpu.async_copy(fbiasT_hbm.at[0].at[fidx_v], fb_v, sem_fb)

        pend = []
        for kk in range(D):
            cu = pltpu.async_copy(
                uembT_hbm.at[kk].at[uidx_v], urows_v.at[kk], sem_u)
            cf = pltpu.async_copy(
                fembT_hbm.at[kk].at[fidx_v], frows_v.at[kk], sem_f)
            pend.append(cu)
            pend.append(cf)
        for c in pend:
            c.wait()

        zero = jnp.zeros((L,), jnp.float32)

        def dim_body(kk, accs):
            a0, a1, a2, a3 = accs
            for j in range(0, BW // L, 4):
                a0 = a0 + (urows_v[kk, pl.ds((j + 0) * L, L)]
                           * frows_v[kk, pl.ds((j + 0) * L, L)])
                a1 = a1 + (urows_v[kk, pl.ds((j + 1) * L, L)]
                           * frows_v[kk, pl.ds((j + 1) * L, L)])
                a2 = a2 + (urows_v[kk, pl.ds((j + 2) * L, L)]
                           * frows_v[kk, pl.ds((j + 2) * L, L)])
                a3 = a3 + (urows_v[kk, pl.ds((j + 3) * L, L)]
                           * frows_v[kk, pl.ds((j + 3) * L, L)])
            return (a0, a1, a2, a3)

        a0, a1, a2, a3 = lax.fori_loop(0, D, dim_body, (zero, zero, zero, zero))
        part_v[...] = (a0 + a1) + (a2 + a3)
        pltpu.sync_copy(part_v, part_hbm.at[pl.ds(wid * L, L)])
        cub.wait()
        cfb.wait()
        pltpu.sync_copy(ub_v, ub_hbm.at[pl.ds(base, BW)])
        pltpu.sync_copy(fb_v, fb_hbm.at[pl.ds(base, BW)])

    return k(uidx, fidx, uembT, ubiasT, fembT, fbiasT)


def _tc_finish(part, ub, fb):
    def body(p_ref, u_ref, f_ref, o_ref):
        s = jnp.sum(p_ref[...])
        o_ref[...] = jax.nn.sigmoid(u_ref[...] + f_ref[...] + s)

    return pl.pallas_call(
        body,
        out_shape=jax.ShapeDtypeStruct((128, 128), jnp.float32),
    )(part.reshape(4, 128), ub.reshape(128, 128), fb.reshape(128, 128))


def kernel(inputs, user_emb, user_bias, food_emb, food_bias):
    uidx = inputs[:, 0]
    fidx = inputs[:, 1]
    ulin, flin = _sc_detile(user_emb.T, food_emb.T)
    part, ub, fb = _sc_gather_partials(
        uidx, fidx, ulin.reshape(D, 1000000), user_bias.T,
        flin.reshape(D, 1000000), food_bias.T)
    return _tc_finish(part, ub, fb).reshape(B, 1)


# speedup vs baseline: 22.9232x; 1.0006x over previous
import functools

import jax
import jax.numpy as jnp
from jax import lax
from jax.experimental import pallas as pl
from jax.experimental.pallas import tpu as pltpu
from jax.experimental.pallas import tpu_sc as plsc

B = 16384
D = 64
NC = 2
NS = 16
NW = NC * NS
BW = B // NW
L = 16


def _sc_detile(uembT, fembT):
    mesh = plsc.VectorSubcoreMesh(core_axis_name="c", subcore_axis_name="s")

    NB = 4
    NP = 32
    PLEN = 31232
    PLAST = 1000000 - 31 * PLEN
    pieces = [(p * PLEN, PLEN if p < NP - 1 else PLAST) for p in range(NP)]

    @functools.partial(
        pl.kernel,
        mesh=mesh,
        out_type=(
            jax.ShapeDtypeStruct((D * 1000000,), jnp.float32),
            jax.ShapeDtypeStruct((D * 1000000,), jnp.float32),
        ),
        scratch_types=(
            [pltpu.VMEM((PLAST,), jnp.float32) for _ in range(NB)]
            + [pltpu.SemaphoreType.DMA for _ in range(2 * NB)]
        ),
    )
    def k(uembT_hbm, fembT_hbm, ulin_hbm, flin_hbm, *scr):
        bufs = scr[:NB]
        rsems = scr[NB:2 * NB]
        wsems = scr[2 * NB:]
        wid = lax.axis_index("s") * NC + lax.axis_index("c")
        jobs = []
        for r in range(2):
            jobs += [("u", r, p) for p in range(NP)]
            jobs += [("f", r, p) for p in range(NP)]

        def fire_read(j):
            tbl, r, p = jobs[j]
            src = uembT_hbm if tbl == "u" else fembT_hbm
            off, ln = pieces[p]
            b = j % NB
            return pltpu.async_copy(
                src.at[wid * 2 + r].at[pl.ds(off, ln)],
                bufs[b].at[pl.ds(0, ln)], rsems[b])

        def fire_write(j):
            tbl, r, p = jobs[j]
            dst = ulin_hbm if tbl == "u" else flin_hbm
            off, ln = pieces[p]
            b = j % NB
            return pltpu.async_copy(
                bufs[b].at[pl.ds(0, ln)],
                dst.at[pl.ds((wid * 2 + r) * 1000000 + off, ln)], wsems[b])

        nj = len(jobs)
        AH = 2
        rd = [None] * nj
        wr = [None] * nj
        for j in range(AH):
            rd[j] = fire_read(j)
        for j in range(nj):
            rd[j].wait()
            wr[j] = fire_write(j)
            if j + AH < nj:
                if j - (NB - AH) >= 0:
                    wr[j - (NB - AH)].wait()
                rd[j + AH] = fire_read(j + AH)
        for j in range(nj - NB, nj):
            if 0 <= j:
                wr[j].wait()

    return k(uembT, fembT)


def _sc_gather_partials(uidx, fidx, uembT, ubiasT, fembT, fbiasT):
    mesh = plsc.VectorSubcoreMesh(core_axis_name="c", subcore_axis_name="s")

    @functools.partial(
        pl.kernel,
        mesh=mesh,
        compiler_params=pltpu.CompilerParams(use_tc_tiling_on_sc=False),
        out_type=(
            jax.ShapeDtypeStruct((NW * L,), jnp.float32),
            jax.ShapeDtypeStruct((B,), jnp.float32),
            jax.ShapeDtypeStruct((B,), jnp.float32),
        ),
        scratch_types=[
            pltpu.VMEM((BW,), jnp.int32),
            pltpu.VMEM((BW,), jnp.int32),
            pltpu.VMEM((D, BW), jnp.float32),
            pltpu.VMEM((D, BW), jnp.float32),
            pltpu.VMEM((BW,), jnp.float32),
            pltpu.VMEM((BW,), jnp.float32),
            pltpu.VMEM((L,), jnp.float32),
            pltpu.SemaphoreType.DMA,
            pltpu.SemaphoreType.DMA,
            pltpu.SemaphoreType.DMA,
            pltpu.SemaphoreType.DMA,
        ],
    )
    def k(uidx_hbm, fidx_hbm, uembT_hbm, ubiasT_hbm, fembT_hbm, fbiasT_hbm,
          part_hbm, ub_hbm, fb_hbm,
          uidx_v, fidx_v, urows_v, frows_v, ub_v, fb_v, part_v,
          sem_u, sem_f, sem_ub, sem_fb):
        wid = lax.axis_index("s") * NC + lax.axis_index("c")
        base = wid * BW
        pltpu.sync_copy(uidx_hbm.at[pl.ds(base, BW)], uidx_v)
        pltpu.sync_copy(fidx_hbm.at[pl.ds(base, BW)], fidx_v)
        cub = pltpu.async_copy(ubiasT_hbm.at[0].at[uidx_v], ub_v, sem_ub)
        cfb = pltpu.async_copy(fbiasT_hbm.at[0].at[fidx_v], fb_v, sem_fb)

        pend = []
        for kk in range(D):
            cu = pltpu.async_copy(
                uembT_hbm.at[kk].at[uidx_v], urows_v.at[kk], sem_u)
            cf = pltpu.async_copy(
                fembT_hbm.at[kk].at[fidx_v], frows_v.at[kk], sem_f)
            pend.append(cu)
            pend.append(cf)
        for c in pend:
            c.wait()

        zero = jnp.zeros((L,), jnp.float32)

        def dim_body(kk, accs):
            a0, a1, a2, a3 = accs
            for j in range(0, BW // L, 4):
                a0 = a0 + (urows_v[kk, pl.ds((j + 0) * L, L)]
                           * frows_v[kk, pl.ds((j + 0) * L, L)])
                a1 = a1 + (urows_v[kk, pl.ds((j + 1) * L, L)]
                           * frows_v[kk, pl.ds((j + 1) * L, L)])
                a2 = a2 + (urows_v[kk, pl.ds((j + 2) * L, L)]
                           * frows_v[kk, pl.ds((j + 2) * L, L)])
                a3 = a3 + (urows_v[kk, pl.ds((j + 3) * L, L)]
                           * frows_v[kk, pl.ds((j + 3) * L, L)])
            return (a0, a1, a2, a3)

        a0, a1, a2, a3 = lax.fori_loop(0, D, dim_body, (zero, zero, zero, zero))
        part_v[...] = (a0 + a1) + (a2 + a3)
        pltpu.sync_copy(part_v, part_hbm.at[pl.ds(wid * L, L)])
        cub.wait()
        cfb.wait()
        pltpu.sync_copy(ub_v, ub_hbm.at[pl.ds(base, BW)])
        pltpu.sync_copy(fb_v, fb_hbm.at[pl.ds(base, BW)])

    return k(uidx, fidx, uembT, ubiasT, fembT, fbiasT)


def _tc_finish(part, ub, fb):
    def body(p_ref, u_ref, f_ref, o_ref):
        s = jnp.sum(p_ref[...])
        o_ref[...] = jax.nn.sigmoid(u_ref[...] + f_ref[...] + s)

    return pl.pallas_call(
        body,
        out_shape=jax.ShapeDtypeStruct((128, 128), jnp.float32),
    )(part.reshape(4, 128), ub.reshape(128, 128), fb.reshape(128, 128))


def kernel(inputs, user_emb, user_bias, food_emb, food_bias):
    uidx = inputs[:, 0]
    fidx = inputs[:, 1]
    ulin, flin = _sc_detile(user_emb.T, food_emb.T)
    part, ub, fb = _sc_gather_partials(
        uidx, fidx, ulin.reshape(D, 1000000), user_bias.T,
        flin.reshape(D, 1000000), food_bias.T)
    return _tc_finish(part, ub, fb).reshape(B, 1)
